# baseline (speedup 1.0000x reference)
import functools

import jax
import jax.numpy as jnp
from jax import lax
from jax.experimental import pallas as pl
from jax.experimental.pallas import tpu as pltpu
from jax.experimental.pallas import tpu_sc as plsc

N = 10000
E = 320000
IN_DIM = 128
HID = 256
OUT_DIM = 128
F = 128
FQ = 64
NQ = 4
NPASS = 2

NC = 2
NS = 16
NW = NC * NS

K = 128
NP = 10240
ROWS_PER_TILE = NP // NS
EPT_AGG = 20480
EP = NS * EPT_AGG
NB_AGG = EPT_AGG // K
EPT_DEG = EP // NW
NB_DEG = EPT_DEG // K
DEGW = 16

_MESH = plsc.VectorSubcoreMesh(core_axis_name="c", subcore_axis_name="s")
_SC_PARAMS = pltpu.CompilerParams(use_tc_tiling_on_sc=False)
_PREC = jax.lax.Precision.DEFAULT


def _elu(x):
    return jnp.where(x > 0, x, jnp.exp(jnp.minimum(x, 0.0)) - 1.0)


@functools.partial(
    pl.kernel,
    out_type=jax.ShapeDtypeStruct((NC * NP, DEGW), jnp.float32),
    mesh=_MESH,
    compiler_params=_SC_PARAMS,
    scratch_types=[
        pltpu.VMEM((NB_DEG, K), jnp.int32),
        pltpu.VMEM((K, DEGW), jnp.float32),
        pltpu.VMEM_SHARED((NP, DEGW), jnp.float32),
        [pltpu.SemaphoreType.DMA] * 8,
    ],
)
def _degree_kernel(dst_hbm, zeros_hbm, ones_hbm, out_hbm, dstv, ones_v,
                   deg_sp, deg_sem):
    cid = lax.axis_index("c")
    sid = lax.axis_index("s")
    wid = cid * NS + sid

    pltpu.sync_copy(zeros_hbm.at[pl.ds(sid * ROWS_PER_TILE, ROWS_PER_TILE)],
                    deg_sp.at[pl.ds(sid * ROWS_PER_TILE, ROWS_PER_TILE)])

    pltpu.sync_copy(dst_hbm.at[wid], dstv)
    pltpu.sync_copy(ones_hbm, ones_v)
    plsc.subcore_barrier()

    def fire8(i, _):
        base = 8 * i
        for r in range(8):
            pltpu.async_copy(ones_v, deg_sp.at[dstv.at[base + r]],
                             deg_sem[r], add=True)
        for r in range(8):
            pltpu.make_async_copy(ones_v, deg_sp.at[dstv.at[base + r]],
                                  deg_sem[r]).wait()
        return 0
    lax.fori_loop(0, NB_DEG // 8, fire8, 0)

    plsc.subcore_barrier()
    pltpu.sync_copy(deg_sp.at[pl.ds(sid * ROWS_PER_TILE, ROWS_PER_TILE)],
                    out_hbm.at[pl.ds(cid * NP + sid * ROWS_PER_TILE,
                                     ROWS_PER_TILE)])


@functools.partial(
    pl.kernel,
    out_type=jax.ShapeDtypeStruct((NQ * NP, FQ), jnp.float32),
    mesh=_MESH,
    compiler_params=_SC_PARAMS,
    scratch_types=[
        pltpu.VMEM((NB_AGG, K), jnp.int32),
        pltpu.VMEM((NB_AGG, K), jnp.int32),
        pltpu.VMEM((5, K, FQ), jnp.float32),
        pltpu.VMEM_SHARED((NP, FQ), jnp.float32),
        [pltpu.SemaphoreType.DMA] * 5,
        [pltpu.SemaphoreType.DMA] * 5,
    ],
)
def _aggregate_kernel(src_hbm, dst_hbm, g_hbm, zeros_hbm, out_hbm,
                      srcv, dstv, rows, acc_sp, sem_g, sem_s):
    cid = lax.axis_index("c")
    sid = lax.axis_index("s")

    pltpu.sync_copy(dst_hbm.at[sid], dstv)
    pltpu.sync_copy(src_hbm.at[sid], srcv)

    for p in range(NPASS):
        q = p * NC + cid

        pltpu.sync_copy(
            zeros_hbm.at[pl.ds(sid * ROWS_PER_TILE, ROWS_PER_TILE)],
            acc_sp.at[pl.ds(sid * ROWS_PER_TILE, ROWS_PER_TILE)])
        plsc.subcore_barrier()

        gq = g_hbm.at[q]
        for r in range(5):
            pltpu.async_copy(gq.at[srcv.at[r]], rows.at[r], sem_g[r])

        def quint(i, _):
            base = 5 * i
            for r in range(5):
                b = base + r
                pltpu.make_async_copy(gq.at[srcv.at[b]], rows.at[r],
                                      sem_g[r]).wait()
                pltpu.async_copy(rows.at[r], acc_sp.at[dstv.at[b]],
                                 sem_s[r], add=True)
            for r in range(5):
                b = base + r
                pltpu.make_async_copy(rows.at[r], acc_sp.at[dstv.at[b]],
                                      sem_s[r]).wait()

                @pl.when(b + 5 < NB_AGG)
                def _():
                    pltpu.async_copy(gq.at[srcv.at[b + 5]], rows.at[r],
                                     sem_g[r])

            return 0
        lax.fori_loop(0, NB_AGG // 5, quint, 0)

        plsc.subcore_barrier()
        pltpu.sync_copy(acc_sp.at[pl.ds(sid * ROWS_PER_TILE, ROWS_PER_TILE)],
                        out_hbm.at[pl.ds(q * NP + sid * ROWS_PER_TILE,
                                         ROWS_PER_TILE)])


NBLK = 1000
GRID = N // NBLK


def _mlp_main_body(x_ref, w1_ref, b1_ref, w2_ref, b2_ref, h_ref):
    x = x_ref[...]
    h = jnp.dot(x, w1_ref[...], precision=_PREC) + b1_ref[...]
    h = _elu(h)
    h = jnp.dot(h, w2_ref[...], precision=_PREC) + b2_ref[...]
    h_ref[...] = _elu(h)


def _mlp_main_call(x, W1, b1r, W2, b2r):
    return pl.pallas_call(
        _mlp_main_body,
        grid=(GRID,),
        in_specs=[
            pl.BlockSpec((NBLK, IN_DIM), lambda i: (i, 0)),
            pl.BlockSpec((IN_DIM, HID), lambda i: (0, 0)),
            pl.BlockSpec((1, HID), lambda i: (0, 0)),
            pl.BlockSpec((HID, HID), lambda i: (0, 0)),
            pl.BlockSpec((1, HID), lambda i: (0, 0)),
        ],
        out_specs=pl.BlockSpec((NBLK, HID), lambda i: (i, 0)),
        out_shape=jax.ShapeDtypeStruct((N, HID), jnp.float32),
    )(x, W1, b1r, W2, b2r)


def _gproj_body(h_ref, wg_ref, degp_ref, g_ref):
    deg = jnp.sum(degp_ref[...], axis=(0, 2)) * (1.0 / DEGW) + 1.0
    dinv = jax.lax.rsqrt(deg)[:, None]
    g = jnp.dot(h_ref[...] * dinv, wg_ref[...], precision=_PREC)
    for q in range(NQ):
        g_ref[q] = g[:, q * FQ:(q + 1) * FQ]


def _gproj_call(h, Wg, degp):
    return pl.pallas_call(
        _gproj_body,
        grid=(GRID,),
        in_specs=[
            pl.BlockSpec((NBLK, HID), lambda i: (i, 0)),
            pl.BlockSpec((HID, HID), lambda i: (0, 0)),
            pl.BlockSpec((NC, NBLK, DEGW), lambda i: (0, i, 0)),
        ],
        out_specs=pl.BlockSpec((NQ, NBLK, FQ), lambda i: (0, i, 0)),
        out_shape=jax.ShapeDtypeStruct((NQ, N, FQ), jnp.float32),
    )(h, Wg, degp)


def _head_body(acc_ref, g_ref, degp_ref, bg_ref, wf_ref, bf_ref, y_ref):
    deg = jnp.sum(degp_ref[...], axis=(0, 2)) * (1.0 / DEGW) + 1.0
    dinv = jax.lax.rsqrt(deg)[:, None]
    bg = bg_ref[...]
    parts = [
        _elu((g_ref[q] + acc_ref[q]) * dinv + bg[:, q * FQ:(q + 1) * FQ])
        for q in range(NQ)
    ]
    h = jnp.concatenate(parts, axis=1)
    y_ref[...] = jnp.dot(h, wf_ref[...], precision=_PREC) + bf_ref[...]


def _head_call(acc, g, degp, bgr, Wf, bfr):
    return pl.pallas_call(
        _head_body,
        grid=(GRID,),
        in_specs=[
            pl.BlockSpec((NQ, NBLK, FQ), lambda i: (0, i, 0)),
            pl.BlockSpec((NQ, NBLK, FQ), lambda i: (0, i, 0)),
            pl.BlockSpec((NC, NBLK, DEGW), lambda i: (0, i, 0)),
            pl.BlockSpec((1, HID), lambda i: (0, 0)),
            pl.BlockSpec((HID, OUT_DIM), lambda i: (0, 0)),
            pl.BlockSpec((1, OUT_DIM), lambda i: (0, 0)),
        ],
        out_specs=pl.BlockSpec((NBLK, OUT_DIM), lambda i: (i, 0)),
        out_shape=jax.ShapeDtypeStruct((N, OUT_DIM), jnp.float32),
    )(acc, g, degp, bgr, Wf, bfr)


def kernel(x, edge_index, W1, b1, W2, b2, Wg, bg, Wf, bf):
    ei = edge_index.astype(jnp.int32)
    src2 = ei[0].reshape(E // K, K)
    dst2 = ei[1].reshape(E // K, K)

    pad = EP - E
    pad_src = (jnp.arange(pad, dtype=jnp.int32) % N).reshape(pad // K, K)
    pad_dst = (N + jnp.arange(pad, dtype=jnp.int32) % (NP - N)).reshape(
        pad // K, K)
    srcp = jnp.concatenate([src2, pad_src])
    dstp = jnp.concatenate([dst2, pad_dst])

    src_agg = srcp.reshape(NS, NB_AGG, K)
    dst_agg = dstp.reshape(NS, NB_AGG, K)
    dst_deg = dstp.reshape(NW, NB_DEG, K)

    zeros_deg = jnp.zeros((NP, DEGW), jnp.float32)
    zeros_f = jnp.zeros((NP, FQ), jnp.float32)

    b1r = b1.reshape(1, HID)
    b2r = b2.reshape(1, HID)
    bgr = bg.reshape(1, HID)
    bfr = bf.reshape(1, OUT_DIM)

    ones_rows = jnp.ones((K, DEGW), jnp.float32)
    degp = _degree_kernel(dst_deg, zeros_deg, ones_rows).reshape(NC, NP, DEGW)
    h2 = _mlp_main_call(x, W1, b1r, W2, b2r)

    g = _gproj_call(h2, Wg, degp)
    acc = _aggregate_kernel(src_agg, dst_agg, g, zeros_f)
    y = _head_call(acc.reshape(NQ, NP, FQ), g, degp, bgr, Wf, bfr)
    return y


# speedup vs baseline: 1.0076x; 1.0076x over previous
import functools

import jax
import jax.numpy as jnp
from jax import lax
from jax.experimental import pallas as pl
from jax.experimental.pallas import tpu as pltpu
from jax.experimental.pallas import tpu_sc as plsc

N = 10000
E = 320000
IN_DIM = 128
HID = 256
OUT_DIM = 128
F = 128
FQ = 64
NQ = 4
NPASS = 2

NC = 2
NS = 16
NW = NC * NS

K = 128
NP = 10240
ROWS_PER_TILE = NP // NS
EPT_AGG = 20480
EP = NS * EPT_AGG
NB_AGG = EPT_AGG // K
EPT_DEG = EP // NW
NB_DEG = EPT_DEG // K
DEGW = 16

_MESH = plsc.VectorSubcoreMesh(core_axis_name="c", subcore_axis_name="s")
_SC_PARAMS = pltpu.CompilerParams(use_tc_tiling_on_sc=False)
_PREC = jax.lax.Precision.DEFAULT


def _elu(x):
    return jnp.where(x > 0, x, jnp.exp(jnp.minimum(x, 0.0)) - 1.0)


@functools.partial(
    pl.kernel,
    out_type=jax.ShapeDtypeStruct((NC * NP, DEGW), jnp.float32),
    mesh=_MESH,
    compiler_params=_SC_PARAMS,
    scratch_types=[
        pltpu.VMEM((NB_DEG, K), jnp.int32),
        pltpu.VMEM((K, DEGW), jnp.float32),
        pltpu.VMEM_SHARED((NP, DEGW), jnp.float32),
        [pltpu.SemaphoreType.DMA] * 8,
    ],
)
def _degree_kernel(dst_hbm, zeros_hbm, ones_hbm, out_hbm, dstv, ones_v,
                   deg_sp, deg_sem):
    cid = lax.axis_index("c")
    sid = lax.axis_index("s")
    wid = cid * NS + sid

    pltpu.sync_copy(zeros_hbm.at[pl.ds(sid * ROWS_PER_TILE, ROWS_PER_TILE)],
                    deg_sp.at[pl.ds(sid * ROWS_PER_TILE, ROWS_PER_TILE)])

    pltpu.sync_copy(dst_hbm.at[wid], dstv)
    pltpu.sync_copy(ones_hbm, ones_v)
    plsc.subcore_barrier()

    def fire8(i, _):
        base = 8 * i
        for r in range(8):
            pltpu.async_copy(ones_v, deg_sp.at[dstv.at[base + r]],
                             deg_sem[r], add=True)
        for r in range(8):
            pltpu.make_async_copy(ones_v, deg_sp.at[dstv.at[base + r]],
                                  deg_sem[r]).wait()
        return 0
    lax.fori_loop(0, NB_DEG // 8, fire8, 0)

    plsc.subcore_barrier()
    pltpu.sync_copy(deg_sp.at[pl.ds(sid * ROWS_PER_TILE, ROWS_PER_TILE)],
                    out_hbm.at[pl.ds(cid * NP + sid * ROWS_PER_TILE,
                                     ROWS_PER_TILE)])


@functools.partial(
    pl.kernel,
    out_type=jax.ShapeDtypeStruct((NQ * NP, FQ), jnp.float32),
    mesh=_MESH,
    compiler_params=_SC_PARAMS,
    scratch_types=[
        pltpu.VMEM((NB_AGG, K), jnp.int32),
        pltpu.VMEM((NB_AGG, K), jnp.int32),
        pltpu.VMEM((5, K, FQ), jnp.float32),
        pltpu.VMEM_SHARED((NP, FQ), jnp.float32),
        [pltpu.SemaphoreType.DMA] * 5,
        [pltpu.SemaphoreType.DMA] * 5,
    ],
)
def _aggregate_kernel(src_hbm, dst_hbm, g_hbm, zeros_hbm, out_hbm,
                      srcv, dstv, rows, acc_sp, sem_g, sem_s):
    cid = lax.axis_index("c")
    sid = lax.axis_index("s")

    pltpu.sync_copy(dst_hbm.at[sid], dstv)
    pltpu.sync_copy(src_hbm.at[sid], srcv)

    for p in range(NPASS):
        q = p * NC + cid

        pltpu.sync_copy(
            zeros_hbm.at[pl.ds(sid * ROWS_PER_TILE, ROWS_PER_TILE)],
            acc_sp.at[pl.ds(sid * ROWS_PER_TILE, ROWS_PER_TILE)])
        plsc.subcore_barrier()

        gq = g_hbm.at[q]
        for r in range(5):
            pltpu.async_copy(gq.at[srcv.at[r]], rows.at[r], sem_g[r])

        def quint(i, _):
            base = 5 * i
            for r in range(5):
                b = base + r
                pltpu.make_async_copy(gq.at[srcv.at[b]], rows.at[r],
                                      sem_g[r]).wait()
                pltpu.async_copy(rows.at[r], acc_sp.at[dstv.at[b]],
                                 sem_s[r], add=True)
            for r in range(5):
                b = base + r
                pltpu.make_async_copy(rows.at[r], acc_sp.at[dstv.at[b]],
                                      sem_s[r]).wait()

                @pl.when(b + 5 < NB_AGG)
                def _():
                    pltpu.async_copy(gq.at[srcv.at[b + 5]], rows.at[r],
                                     sem_g[r])

            return 0
        lax.fori_loop(0, NB_AGG // 5, quint, 0)

        plsc.subcore_barrier()
        pltpu.sync_copy(acc_sp.at[pl.ds(sid * ROWS_PER_TILE, ROWS_PER_TILE)],
                        out_hbm.at[pl.ds(q * NP + sid * ROWS_PER_TILE,
                                         ROWS_PER_TILE)])


NBLK = 1000
GRID = N // NBLK


def _mlp_body(x_ref, w1_ref, b1_ref, w2_ref, b2_ref, wg_ref, degp_ref, g_ref):
    x = x_ref[...]
    h = jnp.dot(x, w1_ref[...], precision=_PREC) + b1_ref[...]
    h = _elu(h)
    h = jnp.dot(h, w2_ref[...], precision=_PREC) + b2_ref[...]
    h = _elu(h)
    hg = jnp.dot(h, wg_ref[...], precision=_PREC)
    deg = jnp.sum(degp_ref[...], axis=(0, 2)) * (1.0 / DEGW) + 1.0
    dinv = jax.lax.rsqrt(deg)[:, None]
    g = hg * dinv
    for q in range(NQ):
        g_ref[q] = g[:, q * FQ:(q + 1) * FQ]


def _mlp_call(x, W1, b1r, W2, b2r, Wg, degp):
    return pl.pallas_call(
        _mlp_body,
        grid=(GRID,),
        in_specs=[
            pl.BlockSpec((NBLK, IN_DIM), lambda i: (i, 0)),
            pl.BlockSpec((IN_DIM, HID), lambda i: (0, 0)),
            pl.BlockSpec((1, HID), lambda i: (0, 0)),
            pl.BlockSpec((HID, HID), lambda i: (0, 0)),
            pl.BlockSpec((1, HID), lambda i: (0, 0)),
            pl.BlockSpec((HID, HID), lambda i: (0, 0)),
            pl.BlockSpec((NC, NBLK, DEGW), lambda i: (0, i, 0)),
        ],
        out_specs=pl.BlockSpec((NQ, NBLK, FQ), lambda i: (0, i, 0)),
        out_shape=jax.ShapeDtypeStruct((NQ, N, FQ), jnp.float32),
    )(x, W1, b1r, W2, b2r, Wg, degp)


def _head_body(acc_ref, g_ref, degp_ref, bg_ref, wf_ref, bf_ref, y_ref):
    deg = jnp.sum(degp_ref[...], axis=(0, 2)) * (1.0 / DEGW) + 1.0
    dinv = jax.lax.rsqrt(deg)[:, None]
    bg = bg_ref[...]
    parts = [
        _elu((g_ref[q] + acc_ref[q]) * dinv + bg[:, q * FQ:(q + 1) * FQ])
        for q in range(NQ)
    ]
    h = jnp.concatenate(parts, axis=1)
    y_ref[...] = jnp.dot(h, wf_ref[...], precision=_PREC) + bf_ref[...]


def _head_call(acc, g, degp, bgr, Wf, bfr):
    return pl.pallas_call(
        _head_body,
        grid=(GRID,),
        in_specs=[
            pl.BlockSpec((NQ, NBLK, FQ), lambda i: (0, i, 0)),
            pl.BlockSpec((NQ, NBLK, FQ), lambda i: (0, i, 0)),
            pl.BlockSpec((NC, NBLK, DEGW), lambda i: (0, i, 0)),
            pl.BlockSpec((1, HID), lambda i: (0, 0)),
            pl.BlockSpec((HID, OUT_DIM), lambda i: (0, 0)),
            pl.BlockSpec((1, OUT_DIM), lambda i: (0, 0)),
        ],
        out_specs=pl.BlockSpec((NBLK, OUT_DIM), lambda i: (i, 0)),
        out_shape=jax.ShapeDtypeStruct((N, OUT_DIM), jnp.float32),
    )(acc, g, degp, bgr, Wf, bfr)


def kernel(x, edge_index, W1, b1, W2, b2, Wg, bg, Wf, bf):
    ei = edge_index.astype(jnp.int32)
    src2 = ei[0].reshape(E // K, K)
    dst2 = ei[1].reshape(E // K, K)

    pad = EP - E
    pad_src = (jnp.arange(pad, dtype=jnp.int32) % N).reshape(pad // K, K)
    pad_dst = (N + jnp.arange(pad, dtype=jnp.int32) % (NP - N)).reshape(
        pad // K, K)
    srcp = jnp.concatenate([src2, pad_src])
    dstp = jnp.concatenate([dst2, pad_dst])

    src_agg = srcp.reshape(NS, NB_AGG, K)
    dst_agg = dstp.reshape(NS, NB_AGG, K)
    dst_deg = dstp.reshape(NW, NB_DEG, K)

    zeros_deg = jnp.zeros((NP, DEGW), jnp.float32)
    zeros_f = jnp.zeros((NP, FQ), jnp.float32)

    b1r = b1.reshape(1, HID)
    b2r = b2.reshape(1, HID)
    bgr = bg.reshape(1, HID)
    bfr = bf.reshape(1, OUT_DIM)

    ones_rows = jnp.ones((K, DEGW), jnp.float32)
    degp = _degree_kernel(dst_deg, zeros_deg, ones_rows).reshape(NC, NP, DEGW)

    g = _mlp_call(x, W1, b1r, W2, b2r, Wg, degp)
    acc = _aggregate_kernel(src_agg, dst_agg, g, zeros_f)
    y = _head_call(acc.reshape(NQ, NP, FQ), g, degp, bgr, Wf, bfr)
    return y
